# baseline (speedup 1.0000x reference)
import functools

import jax
import jax.numpy as jnp
from jax import lax
from jax.experimental import pallas as pl
from jax.experimental.pallas import tpu as pltpu
from jax.experimental.pallas import tpu_sc as plsc

G = 8
EPS = 1e-06
L = 16
NW = 32
D = 512
VPR = D // L


def _shuffle(v, idx):
    return v.at[idx].get(mode="promise_in_bounds")


def _process_row(ibuf, obuf, r):
    lane = lax.broadcasted_iota(jnp.int32, (L,), 0)
    vecs = [ibuf[r, pl.ds(i * L, L)] for i in range(VPR)]
    v0 = vecs[0]
    masked = jnp.where(lane < G, v0, -1.0)
    m = masked
    for sh in (1, 2, 4):
        m = jnp.maximum(m, _shuffle(m, lane ^ sh))
    cand = jnp.where((masked == m) & (lane < G), lane, L)
    s = cand
    for sh in (1, 2, 4, 8):
        s = jnp.minimum(s, _shuffle(s, lane ^ sh))
    perm = (lane & ~(G - 1)) | ((lane + s) & (G - 1))

    accs = [None] * 4
    for i in range(VPR):
        sq = vecs[i] * vecs[i]
        a = i % 4
        accs[a] = sq if accs[a] is None else accs[a] + sq
    ssv = (accs[0] + accs[1]) + (accs[2] + accs[3])
    for sh in (1, 2, 4, 8):
        ssv = ssv + _shuffle(ssv, lane ^ sh)
    y = lax.bitcast_convert_type(
        jnp.int32(0x5F3759DF) - (lax.bitcast_convert_type(ssv, jnp.int32) >> 1),
        jnp.float32)
    for _ in range(2):
        y = y * (1.5 - 0.5 * ssv * y * y)
    inv = 1.0 / (ssv * y + EPS)
    for i in range(VPR):
        obuf[r, pl.ds(i * L, L)] = _shuffle(vecs[i], perm) * inv


def _sc_kernel(rows_per_w, ch):
    nchunk = rows_per_w // ch
    n2 = nchunk // 2
    mesh = plsc.VectorSubcoreMesh(core_axis_name="c", subcore_axis_name="s")

    @functools.partial(
        pl.kernel,
        out_type=jax.ShapeDtypeStruct((NW * rows_per_w, D), jnp.float32),
        mesh=mesh,
        scratch_types=[
            pltpu.VMEM((ch, D), jnp.float32),
            pltpu.VMEM((ch, D), jnp.float32),
            pltpu.VMEM((ch, D), jnp.float32),
            pltpu.VMEM((ch, D), jnp.float32),
            pltpu.SemaphoreType.DMA((2,)),
            pltpu.SemaphoreType.DMA((2,)),
        ],
    )
    def k(desc_hbm, out_hbm, ibuf0, ibuf1, obuf0, obuf1, sem_in, sem_out):
        wid = lax.axis_index("s") * 2 + lax.axis_index("c")
        base = wid * rows_per_w
        ibufs = (ibuf0, ibuf1)
        obufs = (obuf0, obuf1)

        def in_copy(c, b):
            return pltpu.make_async_copy(
                desc_hbm.at[pl.ds(base + c * ch, ch)], ibufs[b], sem_in.at[b])

        def out_copy(c, b):
            return pltpu.make_async_copy(
                obufs[b], out_hbm.at[pl.ds(base + c * ch, ch)], sem_out.at[b])

        in_copy(0, 0).start()
        in_copy(1, 1).start()

        def pair_body(c2, carry):
            for b in range(2):
                c = 2 * c2 + b
                in_copy(c, b).wait()

                @pl.when(c2 > 0)
                def _():
                    out_copy(c - 2, b).wait()

                def row_body(r, rc):
                    _process_row(ibufs[b], obufs[b], r)
                    return rc

                lax.fori_loop(0, ch, row_body, 0)

                @pl.when(c2 < n2 - 1)
                def _():
                    in_copy(c + 2, b).start()

                out_copy(c, b).start()
            return carry

        lax.fori_loop(0, n2, pair_body, 0)
        out_copy(nchunk - 2, 0).wait()
        out_copy(nchunk - 1, 1).wait()

    return k


def kernel(kpts, desc):
    B, K, CG = desc.shape
    rows = B * K
    d2 = desc.reshape(rows, CG)
    out = _sc_kernel(rows // NW, 16)(d2)
    return kpts, out.reshape(B, K, CG)


# speedup vs baseline: 1.1457x; 1.1457x over previous
import functools

import jax
import jax.numpy as jnp
from jax import lax
from jax.experimental import pallas as pl
from jax.experimental.pallas import tpu as pltpu
from jax.experimental.pallas import tpu_sc as plsc

G = 8
EPS = 1e-06
L = 16
NW = 32
D = 512
VPR = D // L


def _shuffle(v, idx):
    return v.at[idx].get(mode="promise_in_bounds")


def _process_row(ibuf, obuf, r):
    lane = lax.broadcasted_iota(jnp.int32, (L,), 0)
    vecs = [ibuf[r, pl.ds(i * L, L)] for i in range(VPR)]
    v0 = vecs[0]
    masked = jnp.where(lane < G, v0, -1.0)
    m = masked
    for sh in (1, 2, 4):
        m = jnp.maximum(m, _shuffle(m, lane ^ sh))
    cand = jnp.where((masked == m) & (lane < G), lane, L)
    s = cand
    for sh in (1, 2, 4, 8):
        s = jnp.minimum(s, _shuffle(s, lane ^ sh))
    perm = (lane & ~(G - 1)) | ((lane + s) & (G - 1))

    accs = [None] * 4
    for i in range(VPR):
        sq = vecs[i] * vecs[i]
        a = i % 4
        accs[a] = sq if accs[a] is None else accs[a] + sq
    ssv = (accs[0] + accs[1]) + (accs[2] + accs[3])
    for sh in (1, 2, 4, 8):
        ssv = ssv + _shuffle(ssv, lane ^ sh)
    y = lax.bitcast_convert_type(
        jnp.int32(0x5F3759DF) - (lax.bitcast_convert_type(ssv, jnp.int32) >> 1),
        jnp.float32)
    for _ in range(2):
        y = y * (1.5 - 0.5 * ssv * y * y)
    inv = 1.0 / (ssv * y + EPS)
    for i in range(VPR):
        obuf[r, pl.ds(i * L, L)] = _shuffle(vecs[i], perm) * inv


def _sc_kernel(rows_per_w, ch):
    nchunk = rows_per_w // ch
    n2 = nchunk // 2
    mesh = plsc.VectorSubcoreMesh(core_axis_name="c", subcore_axis_name="s")

    @functools.partial(
        pl.kernel,
        out_type=jax.ShapeDtypeStruct((NW * rows_per_w, D), jnp.float32),
        mesh=mesh,
        scratch_types=[
            pltpu.VMEM((ch, D), jnp.float32),
            pltpu.VMEM((ch, D), jnp.float32),
            pltpu.VMEM((ch, D), jnp.float32),
            pltpu.VMEM((ch, D), jnp.float32),
            pltpu.SemaphoreType.DMA((2,)),
            pltpu.SemaphoreType.DMA((2,)),
        ],
    )
    def k(desc_hbm, out_hbm, ibuf0, ibuf1, obuf0, obuf1, sem_in, sem_out):
        wid = lax.axis_index("s") * 2 + lax.axis_index("c")
        base = wid * rows_per_w
        ibufs = (ibuf0, ibuf1)
        obufs = (obuf0, obuf1)

        def in_copy(c, b):
            return pltpu.make_async_copy(
                desc_hbm.at[pl.ds(base + c * ch, ch)], ibufs[b], sem_in.at[b])

        def out_copy(c, b):
            return pltpu.make_async_copy(
                obufs[b], out_hbm.at[pl.ds(base + c * ch, ch)], sem_out.at[b])

        in_copy(0, 0).start()
        in_copy(1, 1).start()

        def pair_body(c2, carry):
            for b in range(2):
                c = 2 * c2 + b
                in_copy(c, b).wait()

                @pl.when(c2 > 0)
                def _():
                    out_copy(c - 2, b).wait()

                def row_body(r, rc):
                    _process_row(ibufs[b], obufs[b], r)
                    return rc

                lax.fori_loop(0, ch, row_body, 0)

                @pl.when(c2 < n2 - 1)
                def _():
                    in_copy(c + 2, b).start()

                out_copy(c, b).start()
            return carry

        lax.fori_loop(0, n2, pair_body, 0)
        out_copy(nchunk - 2, 0).wait()
        out_copy(nchunk - 1, 1).wait()

    return k


def kernel(kpts, desc):
    B, K, CG = desc.shape
    rows = B * K
    d2 = desc.reshape(rows, CG)
    out = _sc_kernel(rows // NW, 32)(d2)
    return kpts, out.reshape(B, K, CG)


# speedup vs baseline: 1.1517x; 1.0052x over previous
import functools

import jax
import jax.numpy as jnp
from jax import lax
from jax.experimental import pallas as pl
from jax.experimental.pallas import tpu as pltpu
from jax.experimental.pallas import tpu_sc as plsc

G = 8
EPS = 1e-06
L = 16
NW = 32
D = 512
VPR = D // L


def _shuffle(v, idx):
    return v.at[idx].get(mode="promise_in_bounds")


def _process_row(ibuf, obuf, r):
    lane = lax.broadcasted_iota(jnp.int32, (L,), 0)
    vecs = [ibuf[r, pl.ds(i * L, L)] for i in range(VPR)]
    v0 = vecs[0]
    masked = jnp.where(lane < G, v0, -1.0)
    m = masked
    for sh in (1, 2, 4):
        m = jnp.maximum(m, _shuffle(m, lane ^ sh))
    cand = jnp.where((masked == m) & (lane < G), lane, L)
    s = cand
    for sh in (1, 2, 4, 8):
        s = jnp.minimum(s, _shuffle(s, lane ^ sh))
    perm = (lane & ~(G - 1)) | ((lane + s) & (G - 1))

    accs = [None] * 4
    for i in range(VPR):
        sq = vecs[i] * vecs[i]
        a = i % 4
        accs[a] = sq if accs[a] is None else accs[a] + sq
    ssv = (accs[0] + accs[1]) + (accs[2] + accs[3])
    for sh in (1, 2, 4, 8):
        ssv = ssv + _shuffle(ssv, lane ^ sh)
    y = lax.bitcast_convert_type(
        jnp.int32(0x5F3759DF) - (lax.bitcast_convert_type(ssv, jnp.int32) >> 1),
        jnp.float32)
    for _ in range(2):
        y = y * (1.5 - 0.5 * ssv * y * y)
    inv = 1.0 / (ssv * y + EPS)
    for i in range(VPR):
        obuf[r, pl.ds(i * L, L)] = _shuffle(vecs[i], perm) * inv


def _sc_kernel(rows_per_w, ch):
    nchunk = rows_per_w // ch
    n2 = nchunk // 2
    mesh = plsc.VectorSubcoreMesh(core_axis_name="c", subcore_axis_name="s")

    @functools.partial(
        pl.kernel,
        out_type=jax.ShapeDtypeStruct((NW * rows_per_w, D), jnp.float32),
        mesh=mesh,
        scratch_types=[
            pltpu.VMEM((ch, D), jnp.float32),
            pltpu.VMEM((ch, D), jnp.float32),
            pltpu.VMEM((ch, D), jnp.float32),
            pltpu.VMEM((ch, D), jnp.float32),
            pltpu.SemaphoreType.DMA((2,)),
            pltpu.SemaphoreType.DMA((2,)),
        ],
    )
    def k(desc_hbm, out_hbm, ibuf0, ibuf1, obuf0, obuf1, sem_in, sem_out):
        wid = lax.axis_index("c") * 16 + lax.axis_index("s")
        base = wid * rows_per_w
        ibufs = (ibuf0, ibuf1)
        obufs = (obuf0, obuf1)

        def in_copy(c, b):
            return pltpu.make_async_copy(
                desc_hbm.at[pl.ds(base + c * ch, ch)], ibufs[b], sem_in.at[b])

        def out_copy(c, b):
            return pltpu.make_async_copy(
                obufs[b], out_hbm.at[pl.ds(base + c * ch, ch)], sem_out.at[b])

        in_copy(0, 0).start()
        in_copy(1, 1).start()

        def pair_body(c2, carry):
            for b in range(2):
                c = 2 * c2 + b
                in_copy(c, b).wait()

                @pl.when(c2 > 0)
                def _():
                    out_copy(c - 2, b).wait()

                def row_body(r, rc):
                    _process_row(ibufs[b], obufs[b], r)
                    return rc

                lax.fori_loop(0, ch, row_body, 0)

                @pl.when(c2 < n2 - 1)
                def _():
                    in_copy(c + 2, b).start()

                out_copy(c, b).start()
            return carry

        lax.fori_loop(0, n2, pair_body, 0)
        out_copy(nchunk - 2, 0).wait()
        out_copy(nchunk - 1, 1).wait()

    return k


def kernel(kpts, desc):
    B, K, CG = desc.shape
    rows = B * K
    d2 = desc.reshape(rows, CG)
    out = _sc_kernel(rows // NW, 32)(d2)
    return kpts, out.reshape(B, K, CG)
